# baseline (speedup 1.0000x reference)
import functools

import jax
import jax.numpy as jnp
from jax import lax
from jax.experimental import pallas as pl
from jax.experimental.pallas import tpu as pltpu
from jax.experimental.pallas import tpu_sc as plsc

NUM_EMBEDDINGS = 1000000
EMBED_DIM = 64
DENSE_OUT = 64
BATCH = 16384
HIST = 50

NC = 2
NS = 16
NW = NC * NS

BAGS_PER_TILE = BATCH // NW
CHUNK_BAGS = 4
CHUNKS_PER_TILE = BAGS_PER_TILE // CHUNK_BAGS
GHIST = 56
IDX_PER_CHUNK = CHUNK_BAGS * GHIST
XPAD = 128
SUPER_BAGS = 64
CHUNKS_PER_SUPER = SUPER_BAGS // CHUNK_BAGS
NBUF = 4

_TR_BLOCK = 8192
_TR_GRID = -(-NUM_EMBEDDINGS // _TR_BLOCK)
TP_ROWS = _TR_GRID * _TR_BLOCK
ZPAD_ROWS = TP_ROWS - NUM_EMBEDDINGS


def _tr_body(t_ref, e_ref, o_ref):
    o_ref[:, :EMBED_DIM] = lax.dot_general(
        t_ref[:], e_ref[:], (((0,), (0,)), ((), ())),
        preferred_element_type=jnp.float32).astype(jnp.bfloat16)

    tail = NUM_EMBEDDINGS - (_TR_GRID - 1) * _TR_BLOCK

    @pl.when(pl.program_id(0) == _TR_GRID - 1)
    def _():
        o_ref[pl.ds(tail, _TR_BLOCK - tail), :] = jnp.zeros(
            (_TR_BLOCK - tail, XPAD), jnp.bfloat16)


def _tc_repack(tableT):
    return pl.pallas_call(
        _tr_body,
        grid=(_TR_GRID,),
        in_specs=[pl.BlockSpec((EMBED_DIM, _TR_BLOCK), lambda i: (0, i)),
                  pl.BlockSpec((EMBED_DIM, EMBED_DIM), lambda i: (0, 0))],
        out_specs=pl.BlockSpec((_TR_BLOCK, XPAD), lambda i: (i, 0)),
        out_shape=jax.ShapeDtypeStruct((TP_ROWS, XPAD), jnp.bfloat16),
    )(tableT, jnp.eye(EMBED_DIM, dtype=jnp.float32))


def _sc_gather(xp, table):
    mesh = plsc.VectorSubcoreMesh(core_axis_name="c", subcore_axis_name="s")

    @functools.partial(
        pl.kernel,
        mesh=mesh,
        compiler_params=pltpu.CompilerParams(use_tc_tiling_on_sc=False),
        out_type=jax.ShapeDtypeStruct((BATCH * GHIST, XPAD), jnp.bfloat16),
        scratch_types=[
            pltpu.VMEM((2, SUPER_BAGS, GHIST), jnp.int32),
            pltpu.VMEM((NBUF, IDX_PER_CHUNK, XPAD), jnp.bfloat16),
            pltpu.SemaphoreType.DMA,
            pltpu.SemaphoreType.DMA,
            pltpu.SemaphoreType.DMA,
            pltpu.SemaphoreType.DMA,
            pltpu.SemaphoreType.DMA,
            pltpu.SemaphoreType.DMA,
            pltpu.SemaphoreType.DMA,
            pltpu.SemaphoreType.DMA,
        ],
    )
    def sc_gather(x_hbm, table_hbm, out_hbm, idx_v, rows_v,
                  g0, g1, g2, g3, o0, o1, o2, o3):
        wid = lax.axis_index("s") * NC + lax.axis_index("c")
        bag0 = wid * BAGS_PER_TILE
        gsems = (g0, g1, g2, g3)
        osems = (o0, o1, o2, o3)
        rows_b = tuple(rows_v.at[b] for b in range(NBUF))

        def _bag_idx(chunk, j):
            s = chunk // CHUNKS_PER_SUPER
            r = (chunk % CHUNKS_PER_SUPER) * CHUNK_BAGS + j
            return idx_v.at[s % 2, r]

        def fire_g(chunk, b):
            @pl.when(chunk % CHUNKS_PER_SUPER == 0)
            def _():
                s = chunk // CHUNKS_PER_SUPER
                pltpu.sync_copy(
                    x_hbm.at[pl.ds(bag0 + s * SUPER_BAGS, SUPER_BAGS),
                             pl.ds(0, GHIST)],
                    idx_v.at[s % 2])

            for j in range(CHUNK_BAGS):
                pltpu.async_copy(
                    table_hbm.at[_bag_idx(chunk, j)],
                    rows_b[b].at[pl.ds(j * GHIST, GHIST)],
                    gsems[b])

        def drain_g(chunk, b):
            for j in range(CHUNK_BAGS):
                pltpu.make_async_copy(
                    table_hbm.at[_bag_idx(chunk, j)],
                    rows_b[b].at[pl.ds(j * GHIST, GHIST)],
                    gsems[b]).wait()

        def _out_copy(chunk, j, b):
            bag = bag0 + chunk * CHUNK_BAGS + j
            return pltpu.make_async_copy(
                rows_b[b].at[pl.ds(j * GHIST, GHIST)],
                out_hbm.at[pl.ds(bag * GHIST, GHIST)],
                osems[b])

        def fire_out(chunk, b):
            for j in range(CHUNK_BAGS):
                _out_copy(chunk, j, b).start()

        def drain_out(chunk, b):
            for j in range(CHUNK_BAGS):
                _out_copy(chunk, j, b).wait()

        for b in range(NBUF - 1):
            fire_g(b, b)

        def step(k, carry):
            for r in range(NBUF):
                chunk = NBUF * k + r
                bo = (r + NBUF - 1) % NBUF

                @pl.when(chunk == 0)
                def _():
                    fire_g(NBUF - 1, NBUF - 1)

                @pl.when(jnp.logical_and(chunk >= 1,
                                         chunk + NBUF - 1 < CHUNKS_PER_TILE))
                def _():
                    drain_out(chunk - 1, bo)
                    fire_g(chunk + NBUF - 1, bo)

                drain_g(chunk, r)
                fire_out(chunk, r)
            return carry

        lax.fori_loop(0, CHUNKS_PER_TILE // NBUF, step, 0)
        for t in range(NBUF):
            chunk = CHUNKS_PER_TILE - NBUF + t
            drain_out(chunk, chunk % NBUF)

    return sc_gather(xp, table)


def _tc_body(x_ref, w_ref, b_ref, o_ref):
    x3 = x_ref[:].reshape(_TC_BLOCK, GHIST, XPAD)
    pooled = jnp.sum(x3.astype(jnp.float32), axis=1)
    p = pooled[:, :EMBED_DIM] * (1.0 / HIST)
    logits = lax.dot_general(p, w_ref[:], (((1,), (1,)), ((), ())),
                             preferred_element_type=jnp.float32)
    logits = logits + b_ref[:]
    m = jnp.max(logits, axis=1, keepdims=True)
    e = jnp.exp(logits - m)
    o_ref[:] = e / jnp.sum(e, axis=1, keepdims=True)


_TC_BLOCK = 512


def _tc_dense(gathered, W, b2):
    return pl.pallas_call(
        _tc_body,
        grid=(BATCH // _TC_BLOCK,),
        in_specs=[
            pl.BlockSpec((_TC_BLOCK * GHIST, XPAD), lambda i: (i, 0)),
            pl.BlockSpec((DENSE_OUT, EMBED_DIM), lambda i: (0, 0)),
            pl.BlockSpec((1, DENSE_OUT), lambda i: (0, 0)),
        ],
        out_specs=pl.BlockSpec((_TC_BLOCK, DENSE_OUT), lambda i: (i, 0)),
        out_shape=jax.ShapeDtypeStruct((BATCH, DENSE_OUT), jnp.float32),
    )(gathered, W, b2)


@jax.jit
def kernel(x, table, W, b):
    xi = x.astype(jnp.int32)
    zpad = (NUM_EMBEDDINGS
            + (jnp.arange(BATCH, dtype=jnp.int32)[:, None] * (GHIST - HIST)
               + jnp.arange(GHIST - HIST, dtype=jnp.int32)[None, :])
            % ZPAD_ROWS)
    xp = jnp.pad(jnp.concatenate([xi, zpad], axis=1),
                 ((0, 0), (0, XPAD - GHIST)))
    tp = _tc_repack(table.T)
    g = _sc_gather(xp, tp)
    return _tc_dense(g, W, b.reshape(1, DENSE_OUT))


# speedup vs baseline: 4.1065x; 4.1065x over previous
import functools

import jax
import jax.numpy as jnp
from jax import lax
from jax.experimental import pallas as pl
from jax.experimental.pallas import tpu as pltpu
from jax.experimental.pallas import tpu_sc as plsc

NUM_EMBEDDINGS = 1000000
EMBED_DIM = 64
DENSE_OUT = 64
BATCH = 16384
HIST = 50

NC = 2
NS = 16
NW = NC * NS

BAGS_PER_TILE = BATCH // NW
CHUNK_BAGS = 4
CHUNKS_PER_TILE = BAGS_PER_TILE // CHUNK_BAGS
GHIST = 56
IDX_PER_CHUNK = CHUNK_BAGS * GHIST
XPAD = 128
SUPER_BAGS = 64
CHUNKS_PER_SUPER = SUPER_BAGS // CHUNK_BAGS

_TR_BLOCK = 8192
_TR_GRID = -(-NUM_EMBEDDINGS // _TR_BLOCK)


def _tr_body(t_ref, o_ref):
    eye = jnp.eye(EMBED_DIM, dtype=jnp.float32)
    o_ref[:, :EMBED_DIM] = lax.dot_general(
        t_ref[:], eye, (((0,), (0,)), ((), ())),
        preferred_element_type=jnp.float32)


def _tc_repack(tableT):
    return pl.pallas_call(
        _tr_body,
        grid=(_TR_GRID,),
        in_specs=[pl.BlockSpec((EMBED_DIM, _TR_BLOCK), lambda i: (0, i))],
        out_specs=pl.BlockSpec((_TR_BLOCK, XPAD), lambda i: (i, 0)),
        out_shape=jax.ShapeDtypeStruct((NUM_EMBEDDINGS, XPAD), jnp.float32),
    )(tableT)


def _sc_pool(xp, table):
    mesh = plsc.VectorSubcoreMesh(core_axis_name="c", subcore_axis_name="s")

    @functools.partial(
        pl.kernel,
        mesh=mesh,
        compiler_params=pltpu.CompilerParams(use_tc_tiling_on_sc=False),
        out_type=jax.ShapeDtypeStruct((BATCH, EMBED_DIM), jnp.float32),
        scratch_types=[
            pltpu.VMEM((2, SUPER_BAGS, GHIST), jnp.int32),
            pltpu.VMEM((2, IDX_PER_CHUNK, XPAD), jnp.float32),
            pltpu.VMEM((BAGS_PER_TILE, EMBED_DIM), jnp.float32),
            pltpu.SemaphoreType.DMA,
            pltpu.SemaphoreType.DMA,
        ],
    )
    def sc_pool(x_hbm, table_hbm, out_hbm, idx_v, rows_v, acc_v, sem0, sem1):
        wid = lax.axis_index("s") * NC + lax.axis_index("c")
        bag0 = wid * BAGS_PER_TILE
        sems = (sem0, sem1)
        rows_b = (rows_v.at[0], rows_v.at[1])

        def _bag_idx(chunk, j):
            s = chunk // CHUNKS_PER_SUPER
            r = (chunk % CHUNKS_PER_SUPER) * CHUNK_BAGS + j
            return idx_v.at[s % 2, r]

        def _src(chunk, j):
            return table_hbm.at[_bag_idx(chunk, j)]

        def fire(chunk, b):
            @pl.when(chunk % CHUNKS_PER_SUPER == 0)
            def _():
                s = chunk // CHUNKS_PER_SUPER
                pltpu.sync_copy(
                    x_hbm.at[pl.ds(bag0 + s * SUPER_BAGS, SUPER_BAGS),
                             pl.ds(0, GHIST)],
                    idx_v.at[s % 2])

            for j in range(CHUNK_BAGS):
                pltpu.async_copy(
                    _src(chunk, j),
                    rows_b[b].at[pl.ds(j * GHIST, GHIST)],
                    sems[b])

        def drain(chunk, b):
            for j in range(CHUNK_BAGS):
                pltpu.make_async_copy(
                    _src(chunk, j),
                    rows_b[b].at[pl.ds(j * GHIST, GHIST)],
                    sems[b]).wait()

        def compute(chunk, b):
            rb = rows_b[b]

            def bag_body(j, carry):
                rbase = j * GHIST

                def r_body(ri, accs):
                    out = list(accs)
                    for u in range(10):
                        row = rbase + ri * 10 + u
                        for dk in range(4):
                            out[dk] = out[dk] + rb[row, pl.ds(dk * 16, 16)]
                    return tuple(out)

                z = jnp.zeros((16,), jnp.float32)
                accs = lax.fori_loop(0, HIST // 10, r_body, (z, z, z, z))
                gbag = chunk * CHUNK_BAGS + j
                for dk in range(4):
                    acc_v[gbag, pl.ds(dk * 16, 16)] = accs[dk]
                return carry

            lax.fori_loop(0, CHUNK_BAGS, bag_body, 0)

        fire(0, 0)
        fire(1, 1)

        def step(c, carry):
            for b in range(2):
                chunk = 2 * c + b
                drain(chunk, b)
                compute(chunk, b)

                @pl.when(chunk < CHUNKS_PER_TILE - 2)
                def _():
                    fire(chunk + 2, b)
            return carry

        lax.fori_loop(0, CHUNKS_PER_TILE // 2, step, 0)
        pltpu.sync_copy(acc_v, out_hbm.at[pl.ds(bag0, BAGS_PER_TILE)])

    return sc_pool(xp, table)


def _tc_body(p_ref, w_ref, b_ref, o_ref):
    p = p_ref[:] * (1.0 / HIST)
    logits = lax.dot_general(p, w_ref[:], (((1,), (1,)), ((), ())),
                             preferred_element_type=jnp.float32)
    logits = logits + b_ref[:]
    m = jnp.max(logits, axis=1, keepdims=True)
    e = jnp.exp(logits - m)
    o_ref[:] = e / jnp.sum(e, axis=1, keepdims=True)


_TC_BLOCK = 1024


def _tc_dense(pooled, W, b2):
    return pl.pallas_call(
        _tc_body,
        grid=(BATCH // _TC_BLOCK,),
        in_specs=[
            pl.BlockSpec((_TC_BLOCK, EMBED_DIM), lambda i: (i, 0)),
            pl.BlockSpec((DENSE_OUT, EMBED_DIM), lambda i: (0, 0)),
            pl.BlockSpec((1, DENSE_OUT), lambda i: (0, 0)),
        ],
        out_specs=pl.BlockSpec((_TC_BLOCK, DENSE_OUT), lambda i: (i, 0)),
        out_shape=jax.ShapeDtypeStruct((BATCH, DENSE_OUT), jnp.float32),
    )(pooled, W, b2)


@jax.jit
def kernel(x, table, W, b):
    xi = x.astype(jnp.int32)
    xp = jnp.pad(jnp.concatenate([xi, xi[:, :GHIST - HIST]], axis=1),
                 ((0, 0), (0, XPAD - GHIST)))
    tp = _tc_repack(table.T)
    pooled = _sc_pool(xp, tp)
    return _tc_dense(pooled, W, b.reshape(1, DENSE_OUT))


# speedup vs baseline: 4.2976x; 1.0465x over previous
import functools

import jax
import jax.numpy as jnp
from jax import lax
from jax.experimental import pallas as pl
from jax.experimental.pallas import tpu as pltpu
from jax.experimental.pallas import tpu_sc as plsc

NUM_EMBEDDINGS = 1000000
EMBED_DIM = 64
DENSE_OUT = 64
BATCH = 16384
HIST = 50

NC = 2
NS = 16
NW = NC * NS

BAGS_PER_TILE = BATCH // NW
CHUNK_BAGS = 4
CHUNKS_PER_TILE = BAGS_PER_TILE // CHUNK_BAGS
GHIST = 56
IDX_PER_CHUNK = CHUNK_BAGS * GHIST
XPAD = 128
SUPER_BAGS = 64
CHUNKS_PER_SUPER = SUPER_BAGS // CHUNK_BAGS

_TR_BLOCK = 16384
_TR_GRID = -(-NUM_EMBEDDINGS // _TR_BLOCK)


def _tr_body(t_ref, o_ref):
    eye = jnp.eye(EMBED_DIM, dtype=jnp.float32)
    o_ref[:, :EMBED_DIM] = lax.dot_general(
        t_ref[:], eye, (((0,), (0,)), ((), ())),
        preferred_element_type=jnp.float32)


def _tc_repack(tableT):
    return pl.pallas_call(
        _tr_body,
        grid=(_TR_GRID,),
        in_specs=[pl.BlockSpec((EMBED_DIM, _TR_BLOCK), lambda i: (0, i))],
        out_specs=pl.BlockSpec((_TR_BLOCK, XPAD), lambda i: (i, 0)),
        out_shape=jax.ShapeDtypeStruct((NUM_EMBEDDINGS, XPAD), jnp.float32),
    )(tableT)


def _sc_pool(xp, table):
    mesh = plsc.VectorSubcoreMesh(core_axis_name="c", subcore_axis_name="s")

    @functools.partial(
        pl.kernel,
        mesh=mesh,
        compiler_params=pltpu.CompilerParams(use_tc_tiling_on_sc=False),
        out_type=jax.ShapeDtypeStruct((BATCH, EMBED_DIM), jnp.float32),
        scratch_types=[
            pltpu.VMEM((2, SUPER_BAGS, GHIST), jnp.int32),
            pltpu.VMEM((2, IDX_PER_CHUNK, XPAD), jnp.float32),
            pltpu.VMEM((BAGS_PER_TILE, EMBED_DIM), jnp.float32),
            pltpu.SemaphoreType.DMA,
            pltpu.SemaphoreType.DMA,
        ],
    )
    def sc_pool(x_hbm, table_hbm, out_hbm, idx_v, rows_v, acc_v, sem0, sem1):
        wid = lax.axis_index("s") * NC + lax.axis_index("c")
        bag0 = wid * BAGS_PER_TILE
        sems = (sem0, sem1)
        rows_b = (rows_v.at[0], rows_v.at[1])

        def _bag_idx(chunk, j):
            s = chunk // CHUNKS_PER_SUPER
            r = (chunk % CHUNKS_PER_SUPER) * CHUNK_BAGS + j
            return idx_v.at[s % 2, r]

        def _src(chunk, j):
            return table_hbm.at[_bag_idx(chunk, j)]

        def fire(chunk, b):
            @pl.when(chunk % CHUNKS_PER_SUPER == 0)
            def _():
                s = chunk // CHUNKS_PER_SUPER
                pltpu.sync_copy(
                    x_hbm.at[pl.ds(bag0 + s * SUPER_BAGS, SUPER_BAGS),
                             pl.ds(0, GHIST)],
                    idx_v.at[s % 2])

            for j in range(CHUNK_BAGS):
                pltpu.async_copy(
                    _src(chunk, j),
                    rows_b[b].at[pl.ds(j * GHIST, GHIST)],
                    sems[b])

        def drain(chunk, b):
            for j in range(CHUNK_BAGS):
                pltpu.make_async_copy(
                    _src(chunk, j),
                    rows_b[b].at[pl.ds(j * GHIST, GHIST)],
                    sems[b]).wait()

        def compute(chunk, b):
            rb = rows_b[b]

            def bag_body(j, carry):
                rbase = j * GHIST

                def r_body(ri, accs):
                    out = list(accs)
                    for u in range(10):
                        row = rbase + ri * 10 + u
                        for dk in range(4):
                            out[dk] = out[dk] + rb[row, pl.ds(dk * 16, 16)]
                    return tuple(out)

                z = jnp.zeros((16,), jnp.float32)
                accs = lax.fori_loop(0, HIST // 10, r_body, (z, z, z, z))
                gbag = chunk * CHUNK_BAGS + j
                for dk in range(4):
                    acc_v[gbag, pl.ds(dk * 16, 16)] = accs[dk]
                return carry

            lax.fori_loop(0, CHUNK_BAGS, bag_body, 0)

        fire(0, 0)
        fire(1, 1)

        def step(c, carry):
            for b in range(2):
                chunk = 2 * c + b
                drain(chunk, b)
                compute(chunk, b)

                @pl.when(chunk < CHUNKS_PER_TILE - 2)
                def _():
                    fire(chunk + 2, b)
            return carry

        lax.fori_loop(0, CHUNKS_PER_TILE // 2, step, 0)
        pltpu.sync_copy(acc_v, out_hbm.at[pl.ds(bag0, BAGS_PER_TILE)])

    return sc_pool(xp, table)


def _tc_body(p_ref, w_ref, b_ref, o_ref):
    p = p_ref[:] * (1.0 / HIST)
    logits = lax.dot_general(p, w_ref[:], (((1,), (1,)), ((), ())),
                             preferred_element_type=jnp.float32)
    logits = logits + b_ref[:]
    m = jnp.max(logits, axis=1, keepdims=True)
    e = jnp.exp(logits - m)
    o_ref[:] = e / jnp.sum(e, axis=1, keepdims=True)


_TC_BLOCK = 1024


def _tc_dense(pooled, W, b2):
    return pl.pallas_call(
        _tc_body,
        grid=(BATCH // _TC_BLOCK,),
        in_specs=[
            pl.BlockSpec((_TC_BLOCK, EMBED_DIM), lambda i: (i, 0)),
            pl.BlockSpec((DENSE_OUT, EMBED_DIM), lambda i: (0, 0)),
            pl.BlockSpec((1, DENSE_OUT), lambda i: (0, 0)),
        ],
        out_specs=pl.BlockSpec((_TC_BLOCK, DENSE_OUT), lambda i: (i, 0)),
        out_shape=jax.ShapeDtypeStruct((BATCH, DENSE_OUT), jnp.float32),
    )(pooled, W, b2)


@jax.jit
def kernel(x, table, W, b):
    xi = x.astype(jnp.int32)
    xp = jnp.pad(jnp.concatenate([xi, xi[:, :GHIST - HIST]], axis=1),
                 ((0, 0), (0, XPAD - GHIST)))
    tp = _tc_repack(table.T)
    pooled = _sc_pool(xp, tp)
    return _tc_dense(pooled, W, b.reshape(1, DENSE_OUT))
